# baseline (speedup 1.0000x reference)
import functools

import jax
import jax.numpy as jnp
from jax import lax
from jax.experimental import pallas as pl
from jax.experimental.pallas import tpu as pltpu
from jax.experimental.pallas import tpu_sc as plsc

N = 10000
E = 160000
B = 256
DIN = 15
DE = 4
HID = 30
EDIM = 1024
STEPS = 3
S2S_STEPS = 3
SLOPE = 0.22916667

NC = 2
NS = 16
NWORK = NC * NS
HP = 32
TW = (DE + 1) * HP
C = 128
CPW = 40
EPW = C * CPW
EPAD = EPW * NWORK
NPAD = 10112
RPT = NPAD // NS


def _edge_body(t_hbm, src_hbm, dst_hbm, ea_hbm, zero_hbm, out_hbm,
               s_shared, src_v, dst_v, ea_v, rows_v, msg_v, sem):
    cid = lax.axis_index("c")
    sid = lax.axis_index("s")
    wid = sid * NC + cid

    pltpu.sync_copy(zero_hbm, s_shared.at[pl.ds(sid * RPT, RPT)])
    plsc.subcore_barrier()

    lane = lax.iota(jnp.int32, 16)
    onehot = jnp.where(lane == (HID - 16), 1.0, 0.0).astype(jnp.float32)

    ebase = wid * EPW

    def chunk(k, carry):
        base = ebase + k * C
        pltpu.sync_copy(src_hbm.at[pl.ds(base, C)], src_v)
        pltpu.sync_copy(dst_hbm.at[pl.ds(base, C)], dst_v)
        pltpu.sync_copy(ea_hbm.at[pl.ds(base * DE, C * DE + 16)], ea_v)
        pltpu.async_copy(t_hbm.at[src_v], rows_v, sem).wait()

        def do_edge(i, ev, lo):
            for j in range(2):
                o = j * 16
                m = ((ev[lo + 0] * rows_v[i, pl.ds(o, 16)]
                      + ev[lo + 1] * rows_v[i, pl.ds(HP + o, 16)])
                     + (ev[lo + 2] * rows_v[i, pl.ds(2 * HP + o, 16)]
                        + ev[lo + 3] * rows_v[i, pl.ds(3 * HP + o, 16)])
                     + rows_v[i, pl.ds(4 * HP + o, 16)])
                if j == 1:
                    m = m + onehot
                msg_v[i, pl.ds(o, 16)] = m

        @plsc.parallel_loop(0, C // 2, 1, unroll=4)
        def _edge_loop(i):
            ev = ea_v[pl.ds(2 * DE * i, 16)]
            do_edge(2 * i, ev, 0)
            do_edge(2 * i + 1, ev, DE)
        pltpu.sync_copy(msg_v, s_shared.at[dst_v], add=True)
        return carry

    lax.fori_loop(0, CPW, chunk, 0)
    plsc.subcore_barrier()

    pltpu.sync_copy(s_shared.at[pl.ds(sid * RPT, RPT)],
                    out_hbm.at[cid, pl.ds(sid * RPT, RPT)])


_edge_call = functools.partial(
    pl.kernel,
    out_type=jax.ShapeDtypeStruct((NC, NPAD, HP), jnp.float32),
    mesh=plsc.VectorSubcoreMesh(core_axis_name="c", subcore_axis_name="s"),
    compiler_params=pltpu.CompilerParams(use_tc_tiling_on_sc=False),
    scratch_types=[
        pltpu.VMEM_SHARED((NPAD, HP), jnp.float32),
        pltpu.VMEM((C,), jnp.int32),
        pltpu.VMEM((C,), jnp.int32),
        pltpu.VMEM((C * DE + 16,), jnp.float32),
        pltpu.VMEM((C, TW), jnp.float32),
        pltpu.VMEM((C, HP), jnp.float32),
        pltpu.SemaphoreType.DMA,
    ],
)(_edge_body)


def _rrelu(z):
    return jnp.where(z >= 0, z, z * SLOPE)


def _gru(xg, h, Wih, Whh, bih, bhh):
    gi = xg @ Wih.T + bih
    gh = h @ Whh.T + bhh
    ir, iz, inn = jnp.split(gi, 3, axis=-1)
    hr, hz, hn = jnp.split(gh, 3, axis=-1)
    r = jax.nn.sigmoid(ir + hr)
    z = jax.nn.sigmoid(iz + hz)
    n = jnp.tanh(inn + r * hn)
    return (1.0 - z) * n + z * h


def _lstm(xg, h, c, Wih, Whh, bih, bhh):
    g = xg @ Wih.T + bih + h @ Whh.T + bhh
    i, f, gg, o = jnp.split(g, 4, axis=-1)
    i = jax.nn.sigmoid(i)
    f = jax.nn.sigmoid(f)
    gg = jnp.tanh(gg)
    o = jax.nn.sigmoid(o)
    c2 = f * c + i * gg
    return o * jnp.tanh(c2), c2


def _pmm(a, b):
    return jnp.matmul(a, b, precision=lax.Precision.HIGHEST)


def _set2set(xi, seg_bool, seg_oh, Wih, Whh, bih, bhh):
    h = jnp.zeros((B, HID), jnp.float32)
    c = jnp.zeros((B, HID), jnp.float32)
    q_star = jnp.zeros((B, 2 * HID), jnp.float32)
    for _ in range(S2S_STEPS):
        h, c = _lstm(q_star, h, c, Wih, Whh, bih, bhh)
        e = jnp.sum(xi * _pmm(seg_oh.T, h), axis=-1)
        m = jnp.max(jnp.where(seg_bool, e[None, :], -jnp.inf), axis=1)
        mf = jnp.maximum(m, -1e30)
        ex = jnp.exp(e - _pmm(seg_oh.T, mf))
        ssum = _pmm(seg_oh, ex)
        a = ex / (_pmm(seg_oh.T, ssum) + 1e-16)
        r = _pmm(seg_oh, a[:, None] * xi)
        q_star = jnp.concatenate([h, r], axis=-1)
    return q_star


def kernel(x, edge_index, edge_attr, batch, W0, b0, We, be, Wroot, bconv,
           gWih, gWhh, gbih, gbhh, lWih, lWhh, lbih, lbhh,
           Wflat, bflat, Wout, bout):
    f32 = jnp.float32
    xm = _rrelu(x @ W0 + b0)

    pad_e = EPAD - E
    src_p = jnp.concatenate([edge_index[0], jnp.full((pad_e,), N, jnp.int32)])
    dst_p = jnp.concatenate([edge_index[1], jnp.full((pad_e,), N, jnp.int32)])
    ea_p = jnp.concatenate(
        [edge_attr.reshape(-1), jnp.zeros((pad_e * DE + 16,), f32)])
    zero_blk = jnp.zeros((RPT, HP), f32)

    We3 = We.reshape(DE, HID, HID)
    We3p = jnp.pad(We3, ((0, 0), (0, 0), (0, HP - HID)))
    Mw = We3p.transpose(1, 0, 2).reshape(HID, DE * HP)
    Bp = jnp.pad(be.reshape(HID, HID), ((0, 0), (0, HP - HID)))
    M = jnp.concatenate([Mw, Bp], axis=1)

    h = xm
    for _ in range(STEPS):
        identity = xm
        t = jnp.pad(xm @ M, ((0, NPAD - N), (0, 0)))
        parts = _edge_call(t, src_p, dst_p, ea_p, zero_blk)
        s = parts[0] + parts[1]
        cnt = s[:N, HID]
        agg = s[:N, :HID] / jnp.maximum(cnt, 1.0)[:, None]
        xc = _rrelu(xm @ Wroot + agg + bconv)
        h = _gru(xc, h, gWih, gWhh, gbih, gbhh)
        xm = h + identity

    seg_bool = batch[None, :] == jnp.arange(B, dtype=batch.dtype)[:, None]
    seg_oh = seg_bool.astype(f32)
    q = _set2set(xm, seg_bool, seg_oh, lWih, lWhh, lbih, lbhh)
    cntb = jnp.sum(seg_oh, axis=1)
    ssum = _pmm(seg_oh, xm)
    mean = ssum / jnp.maximum(cntb, 1.0)[:, None]
    mx = jax.ops.segment_max(xm, batch, num_segments=B, indices_are_sorted=True)
    outm = jnp.concatenate([q, mean, mx, ssum], axis=-1)
    outm = _rrelu(outm @ Wflat + bflat)
    out = outm @ Wout + bout
    return out, xm


# speedup vs baseline: 1.1454x; 1.1454x over previous
import functools

import jax
import jax.numpy as jnp
from jax import lax
from jax.experimental import pallas as pl
from jax.experimental.pallas import tpu as pltpu
from jax.experimental.pallas import tpu_sc as plsc

N = 10000
E = 160000
B = 256
DIN = 15
DE = 4
HID = 30
EDIM = 1024
STEPS = 3
S2S_STEPS = 3
SLOPE = 0.22916667

NC = 2
NS = 16
NWORK = NC * NS
HP = 32
TW = (DE + 1) * HP
C = 128
CPW = 40
EPW = C * CPW
EPAD = EPW * NWORK
NPAD = 10112
RPT = NPAD // NS


def _edge_body(t_hbm, src_hbm, dst_hbm, ea_hbm, zero_hbm, out_hbm,
               s_shared, srcall_v, dstall_v, eaall_v,
               rows0_v, rows1_v, msg0_v, msg1_v, sem_g, sem_s):
    cid = lax.axis_index("c")
    sid = lax.axis_index("s")
    wid = sid * NC + cid

    pltpu.sync_copy(zero_hbm, s_shared.at[pl.ds(sid * RPT, RPT)])
    pltpu.sync_copy(src_hbm.at[pl.ds(wid * EPW, EPW)], srcall_v)
    pltpu.sync_copy(dst_hbm.at[pl.ds(wid * CPW, CPW)], dstall_v)
    pltpu.sync_copy(ea_hbm.at[pl.ds(wid * EPW * DE, EPW * DE + 16)], eaall_v)
    plsc.subcore_barrier()

    lane = lax.iota(jnp.int32, 16)
    onehot = jnp.where(lane == (HID - 16), 1.0, 0.0).astype(jnp.float32)

    def gstart(k, rows_r):
        pltpu.async_copy(t_hbm.at[srcall_v.at[pl.ds(k * C, C)]], rows_r, sem_g)

    def gwait(rows_r):
        pltpu.make_async_copy(
            t_hbm.at[srcall_v.at[pl.ds(0, C)]], rows_r, sem_g).wait()

    def sstart(k, msg_r):
        pltpu.async_copy(msg_r, s_shared.at[dstall_v.at[k]], sem_s, add=True)

    def swait(msg_r):
        pltpu.make_async_copy(msg_r, s_shared.at[dstall_v.at[0]], sem_s).wait()

    def compute(k, rows_r, msg_r):
        def do_edge(i, ev, lo):
            for j in range(2):
                o = j * 16
                m = ((ev[lo + 0] * rows_r[i, pl.ds(o, 16)]
                      + ev[lo + 1] * rows_r[i, pl.ds(HP + o, 16)])
                     + (ev[lo + 2] * rows_r[i, pl.ds(2 * HP + o, 16)]
                        + ev[lo + 3] * rows_r[i, pl.ds(3 * HP + o, 16)])
                     + rows_r[i, pl.ds(4 * HP + o, 16)])
                if j == 1:
                    m = m + onehot
                msg_r[i, pl.ds(o, 16)] = m

        @plsc.parallel_loop(0, C // 2, 1, unroll=4)
        def _edge_loop(i):
            ev = eaall_v[pl.ds(k * (C * DE) + 2 * DE * i, 16)]
            do_edge(2 * i, ev, 0)
            do_edge(2 * i + 1, ev, DE)

    gstart(0, rows0_v)

    def k2body(k2, carry):
        k = 2 * k2
        gwait(rows0_v)
        gstart(k + 1, rows1_v)

        @pl.when(k2 > 0)
        def _w0():
            swait(msg0_v)

        compute(k, rows0_v, msg0_v)
        sstart(k, msg0_v)

        gwait(rows1_v)

        @pl.when(k2 < CPW // 2 - 1)
        def _g2():
            gstart(k + 2, rows0_v)

        @pl.when(k2 > 0)
        def _w1():
            swait(msg1_v)

        compute(k + 1, rows1_v, msg1_v)
        sstart(k + 1, msg1_v)
        return carry

    lax.fori_loop(0, CPW // 2, k2body, 0)
    swait(msg0_v)
    swait(msg1_v)
    plsc.subcore_barrier()

    pltpu.sync_copy(s_shared.at[pl.ds(sid * RPT, RPT)],
                    out_hbm.at[cid, pl.ds(sid * RPT, RPT)])


_edge_call = functools.partial(
    pl.kernel,
    out_type=jax.ShapeDtypeStruct((NC, NPAD, HP), jnp.float32),
    mesh=plsc.VectorSubcoreMesh(core_axis_name="c", subcore_axis_name="s"),
    compiler_params=pltpu.CompilerParams(use_tc_tiling_on_sc=False),
    scratch_types=[
        pltpu.VMEM_SHARED((NPAD, HP), jnp.float32),
        pltpu.VMEM((EPW,), jnp.int32),
        pltpu.VMEM((CPW, C), jnp.int32),
        pltpu.VMEM((EPW * DE + 16,), jnp.float32),
        pltpu.VMEM((C, TW), jnp.float32),
        pltpu.VMEM((C, TW), jnp.float32),
        pltpu.VMEM((C, HP), jnp.float32),
        pltpu.VMEM((C, HP), jnp.float32),
        pltpu.SemaphoreType.DMA,
        pltpu.SemaphoreType.DMA,
    ],
)(_edge_body)


def _rrelu(z):
    return jnp.where(z >= 0, z, z * SLOPE)


def _gru(xg, h, Wih, Whh, bih, bhh):
    gi = xg @ Wih.T + bih
    gh = h @ Whh.T + bhh
    ir, iz, inn = jnp.split(gi, 3, axis=-1)
    hr, hz, hn = jnp.split(gh, 3, axis=-1)
    r = jax.nn.sigmoid(ir + hr)
    z = jax.nn.sigmoid(iz + hz)
    n = jnp.tanh(inn + r * hn)
    return (1.0 - z) * n + z * h


def _lstm(xg, h, c, Wih, Whh, bih, bhh):
    g = xg @ Wih.T + bih + h @ Whh.T + bhh
    i, f, gg, o = jnp.split(g, 4, axis=-1)
    i = jax.nn.sigmoid(i)
    f = jax.nn.sigmoid(f)
    gg = jnp.tanh(gg)
    o = jax.nn.sigmoid(o)
    c2 = f * c + i * gg
    return o * jnp.tanh(c2), c2


def _pmm(a, b):
    return jnp.matmul(a, b, precision=lax.Precision.HIGHEST)


def _set2set(xi, seg_bool, seg_oh, Wih, Whh, bih, bhh):
    h = jnp.zeros((B, HID), jnp.float32)
    c = jnp.zeros((B, HID), jnp.float32)
    q_star = jnp.zeros((B, 2 * HID), jnp.float32)
    for _ in range(S2S_STEPS):
        h, c = _lstm(q_star, h, c, Wih, Whh, bih, bhh)
        e = jnp.sum(xi * _pmm(seg_oh.T, h), axis=-1)
        m = jnp.max(jnp.where(seg_bool, e[None, :], -jnp.inf), axis=1)
        mf = jnp.maximum(m, -1e30)
        ex = jnp.exp(e - _pmm(seg_oh.T, mf))
        ssum = _pmm(seg_oh, ex)
        a = ex / (_pmm(seg_oh.T, ssum) + 1e-16)
        r = _pmm(seg_oh, a[:, None] * xi)
        q_star = jnp.concatenate([h, r], axis=-1)
    return q_star


def kernel(x, edge_index, edge_attr, batch, W0, b0, We, be, Wroot, bconv,
           gWih, gWhh, gbih, gbhh, lWih, lWhh, lbih, lbhh,
           Wflat, bflat, Wout, bout):
    with jax.default_matmul_precision("highest"):
        return _impl(x, edge_index, edge_attr, batch, W0, b0, We, be, Wroot,
                     bconv, gWih, gWhh, gbih, gbhh, lWih, lWhh, lbih, lbhh,
                     Wflat, bflat, Wout, bout)


def _impl(x, edge_index, edge_attr, batch, W0, b0, We, be, Wroot, bconv,
          gWih, gWhh, gbih, gbhh, lWih, lWhh, lbih, lbhh,
          Wflat, bflat, Wout, bout):
    f32 = jnp.float32
    xm = _rrelu(x @ W0 + b0)

    pad_e = EPAD - E
    src_p = jnp.concatenate([edge_index[0], jnp.full((pad_e,), N, jnp.int32)])
    dst_p = jnp.concatenate(
        [edge_index[1], jnp.full((pad_e,), N, jnp.int32)]
    ).reshape(NWORK * CPW, C)
    ea_p = jnp.concatenate(
        [edge_attr.reshape(-1), jnp.zeros((pad_e * DE + 16,), f32)])
    zero_blk = jnp.zeros((RPT, HP), f32)

    We3 = We.reshape(DE, HID, HID)
    We3p = jnp.pad(We3, ((0, 0), (0, 0), (0, HP - HID)))
    Mw = We3p.transpose(1, 0, 2).reshape(HID, DE * HP)
    Bp = jnp.pad(be.reshape(HID, HID), ((0, 0), (0, HP - HID)))
    M = jnp.concatenate([Mw, Bp], axis=1)

    h = xm
    for _ in range(STEPS):
        identity = xm
        t = jnp.pad(xm @ M, ((0, NPAD - N), (0, 0)))
        parts = _edge_call(t, src_p, dst_p, ea_p, zero_blk)
        s = parts[0] + parts[1]
        cnt = s[:N, HID]
        agg = s[:N, :HID] / jnp.maximum(cnt, 1.0)[:, None]
        xc = _rrelu(xm @ Wroot + agg + bconv)
        h = _gru(xc, h, gWih, gWhh, gbih, gbhh)
        xm = h + identity

    seg_bool = batch[None, :] == jnp.arange(B, dtype=batch.dtype)[:, None]
    seg_oh = seg_bool.astype(f32)
    q = _set2set(xm, seg_bool, seg_oh, lWih, lWhh, lbih, lbhh)
    cntb = jnp.sum(seg_oh, axis=1)
    ssum = _pmm(seg_oh, xm)
    mean = ssum / jnp.maximum(cntb, 1.0)[:, None]
    mx = jax.ops.segment_max(xm, batch, num_segments=B, indices_are_sorted=True)
    outm = jnp.concatenate([q, mean, mx, ssum], axis=-1)
    outm = _rrelu(outm @ Wflat + bflat)
    out = outm @ Wout + bout
    return out, xm


# speedup vs baseline: 1.1457x; 1.0003x over previous
import functools

import jax
import jax.numpy as jnp
from jax import lax
from jax.experimental import pallas as pl
from jax.experimental.pallas import tpu as pltpu
from jax.experimental.pallas import tpu_sc as plsc

N = 10000
E = 160000
B = 256
DIN = 15
DE = 4
HID = 30
EDIM = 1024
STEPS = 3
S2S_STEPS = 3
SLOPE = 0.22916667

NC = 2
NS = 16
NWORK = NC * NS
HP = 32
TW = (DE + 1) * HP
C = 128
CPW = 40
EPW = C * CPW
EPAD = EPW * NWORK
NPAD = 10112
RPT = NPAD // NS


def _edge_body(t_hbm, src_hbm, dst_hbm, ea_hbm, zero_hbm, out_hbm,
               s_shared, srcall_v, dstall_v, eaall_v,
               rows0_v, rows1_v, msg0_v, msg1_v, sem_g, sem_s):
    cid = lax.axis_index("c")
    sid = lax.axis_index("s")
    wid = sid * NC + cid

    pltpu.sync_copy(zero_hbm, s_shared.at[pl.ds(sid * RPT, RPT)])
    pltpu.sync_copy(src_hbm.at[pl.ds(wid * EPW, EPW)], srcall_v)
    pltpu.sync_copy(dst_hbm.at[pl.ds(wid * CPW, CPW)], dstall_v)
    pltpu.sync_copy(ea_hbm.at[pl.ds(wid * EPW * DE, EPW * DE + 16)], eaall_v)
    plsc.subcore_barrier()

    lane = lax.iota(jnp.int32, 16)
    onehot = jnp.where(lane == (HID - 16), 1.0, 0.0).astype(jnp.float32)

    def gstart(k, rows_r):
        pltpu.async_copy(t_hbm.at[srcall_v.at[pl.ds(k * C, C)]], rows_r, sem_g)

    def gwait(rows_r):
        pltpu.make_async_copy(
            t_hbm.at[srcall_v.at[pl.ds(0, C)]], rows_r, sem_g).wait()

    def sstart(k, msg_r):
        pltpu.async_copy(msg_r, s_shared.at[dstall_v.at[k]], sem_s, add=True)

    def swait(msg_r):
        pltpu.make_async_copy(msg_r, s_shared.at[dstall_v.at[0]], sem_s).wait()

    def compute(k, rows_r, msg_r):
        def do_edge(i, ev, lo):
            for j in range(2):
                o = j * 16
                m = ((ev[lo + 0] * rows_r[i, pl.ds(o, 16)]
                      + ev[lo + 1] * rows_r[i, pl.ds(HP + o, 16)])
                     + (ev[lo + 2] * rows_r[i, pl.ds(2 * HP + o, 16)]
                        + ev[lo + 3] * rows_r[i, pl.ds(3 * HP + o, 16)])
                     + rows_r[i, pl.ds(4 * HP + o, 16)])
                if j == 1:
                    m = m + onehot
                msg_r[i, pl.ds(o, 16)] = m

        @plsc.parallel_loop(0, C // 2, 1, unroll=8)
        def _edge_loop(i):
            ev = eaall_v[pl.ds(k * (C * DE) + 2 * DE * i, 16)]
            do_edge(2 * i, ev, 0)
            do_edge(2 * i + 1, ev, DE)

    gstart(0, rows0_v)

    def k2body(k2, carry):
        k = 2 * k2
        gwait(rows0_v)
        gstart(k + 1, rows1_v)

        @pl.when(k2 > 0)
        def _w0():
            swait(msg0_v)

        compute(k, rows0_v, msg0_v)
        sstart(k, msg0_v)

        gwait(rows1_v)

        @pl.when(k2 < CPW // 2 - 1)
        def _g2():
            gstart(k + 2, rows0_v)

        @pl.when(k2 > 0)
        def _w1():
            swait(msg1_v)

        compute(k + 1, rows1_v, msg1_v)
        sstart(k + 1, msg1_v)
        return carry

    lax.fori_loop(0, CPW // 2, k2body, 0)
    swait(msg0_v)
    swait(msg1_v)
    plsc.subcore_barrier()

    pltpu.sync_copy(s_shared.at[pl.ds(sid * RPT, RPT)],
                    out_hbm.at[cid, pl.ds(sid * RPT, RPT)])


_edge_call = functools.partial(
    pl.kernel,
    out_type=jax.ShapeDtypeStruct((NC, NPAD, HP), jnp.float32),
    mesh=plsc.VectorSubcoreMesh(core_axis_name="c", subcore_axis_name="s"),
    compiler_params=pltpu.CompilerParams(use_tc_tiling_on_sc=False),
    scratch_types=[
        pltpu.VMEM_SHARED((NPAD, HP), jnp.float32),
        pltpu.VMEM((EPW,), jnp.int32),
        pltpu.VMEM((CPW, C), jnp.int32),
        pltpu.VMEM((EPW * DE + 16,), jnp.float32),
        pltpu.VMEM((C, TW), jnp.float32),
        pltpu.VMEM((C, TW), jnp.float32),
        pltpu.VMEM((C, HP), jnp.float32),
        pltpu.VMEM((C, HP), jnp.float32),
        pltpu.SemaphoreType.DMA,
        pltpu.SemaphoreType.DMA,
    ],
)(_edge_body)


def _rrelu(z):
    return jnp.where(z >= 0, z, z * SLOPE)


def _gru(xg, h, Wih, Whh, bih, bhh):
    gi = xg @ Wih.T + bih
    gh = h @ Whh.T + bhh
    ir, iz, inn = jnp.split(gi, 3, axis=-1)
    hr, hz, hn = jnp.split(gh, 3, axis=-1)
    r = jax.nn.sigmoid(ir + hr)
    z = jax.nn.sigmoid(iz + hz)
    n = jnp.tanh(inn + r * hn)
    return (1.0 - z) * n + z * h


def _lstm(xg, h, c, Wih, Whh, bih, bhh):
    g = xg @ Wih.T + bih + h @ Whh.T + bhh
    i, f, gg, o = jnp.split(g, 4, axis=-1)
    i = jax.nn.sigmoid(i)
    f = jax.nn.sigmoid(f)
    gg = jnp.tanh(gg)
    o = jax.nn.sigmoid(o)
    c2 = f * c + i * gg
    return o * jnp.tanh(c2), c2


def _pmm(a, b):
    return jnp.matmul(a, b, precision=lax.Precision.HIGHEST)


def _set2set(xi, seg_bool, seg_oh, Wih, Whh, bih, bhh):
    h = jnp.zeros((B, HID), jnp.float32)
    c = jnp.zeros((B, HID), jnp.float32)
    q_star = jnp.zeros((B, 2 * HID), jnp.float32)
    for _ in range(S2S_STEPS):
        h, c = _lstm(q_star, h, c, Wih, Whh, bih, bhh)
        e = jnp.sum(xi * _pmm(seg_oh.T, h), axis=-1)
        m = jnp.max(jnp.where(seg_bool, e[None, :], -jnp.inf), axis=1)
        mf = jnp.maximum(m, -1e30)
        ex = jnp.exp(e - _pmm(seg_oh.T, mf))
        ssum = _pmm(seg_oh, ex)
        a = ex / (_pmm(seg_oh.T, ssum) + 1e-16)
        r = _pmm(seg_oh, a[:, None] * xi)
        q_star = jnp.concatenate([h, r], axis=-1)
    return q_star


def kernel(x, edge_index, edge_attr, batch, W0, b0, We, be, Wroot, bconv,
           gWih, gWhh, gbih, gbhh, lWih, lWhh, lbih, lbhh,
           Wflat, bflat, Wout, bout):
    with jax.default_matmul_precision("highest"):
        return _impl(x, edge_index, edge_attr, batch, W0, b0, We, be, Wroot,
                     bconv, gWih, gWhh, gbih, gbhh, lWih, lWhh, lbih, lbhh,
                     Wflat, bflat, Wout, bout)


def _impl(x, edge_index, edge_attr, batch, W0, b0, We, be, Wroot, bconv,
          gWih, gWhh, gbih, gbhh, lWih, lWhh, lbih, lbhh,
          Wflat, bflat, Wout, bout):
    f32 = jnp.float32
    xm = _rrelu(x @ W0 + b0)

    pad_e = EPAD - E
    src_p = jnp.concatenate([edge_index[0], jnp.full((pad_e,), N, jnp.int32)])
    dst_p = jnp.concatenate(
        [edge_index[1], jnp.full((pad_e,), N, jnp.int32)]
    ).reshape(NWORK * CPW, C)
    ea_p = jnp.concatenate(
        [edge_attr.reshape(-1), jnp.zeros((pad_e * DE + 16,), f32)])
    zero_blk = jnp.zeros((RPT, HP), f32)

    We3 = We.reshape(DE, HID, HID)
    We3p = jnp.pad(We3, ((0, 0), (0, 0), (0, HP - HID)))
    Mw = We3p.transpose(1, 0, 2).reshape(HID, DE * HP)
    Bp = jnp.pad(be.reshape(HID, HID), ((0, 0), (0, HP - HID)))
    M = jnp.concatenate([Mw, Bp], axis=1)

    h = xm
    for _ in range(STEPS):
        identity = xm
        t = jnp.pad(xm @ M, ((0, NPAD - N), (0, 0)))
        parts = _edge_call(t, src_p, dst_p, ea_p, zero_blk)
        s = parts[0] + parts[1]
        cnt = s[:N, HID]
        agg = s[:N, :HID] / jnp.maximum(cnt, 1.0)[:, None]
        xc = _rrelu(xm @ Wroot + agg + bconv)
        h = _gru(xc, h, gWih, gWhh, gbih, gbhh)
        xm = h + identity

    seg_bool = batch[None, :] == jnp.arange(B, dtype=batch.dtype)[:, None]
    seg_oh = seg_bool.astype(f32)
    q = _set2set(xm, seg_bool, seg_oh, lWih, lWhh, lbih, lbhh)
    cntb = jnp.sum(seg_oh, axis=1)
    ssum = _pmm(seg_oh, xm)
    mean = ssum / jnp.maximum(cntb, 1.0)[:, None]
    mx = jax.ops.segment_max(xm, batch, num_segments=B, indices_are_sorted=True)
    outm = jnp.concatenate([q, mean, mx, ssum], axis=-1)
    outm = _rrelu(outm @ Wflat + bflat)
    out = outm @ Wout + bout
    return out, xm
